# baseline (speedup 1.0000x reference)
import jax
import jax.numpy as jnp
from jax import lax
from jax.experimental import pallas as pl
from jax.experimental.pallas import tpu as pltpu
from jax.experimental.pallas import tpu_sc as plsc

_S = 64
_CHUNK = 64
_NB = 10
_NC = 2
_NS = 16
_NW = _NC * _NS
_SEEDS_PER_W = _S // _NW
_L = 16


def _sc_coeff_body(ls_hbm, ids_hbm, st_hbm, al_hbm, bw2_hbm, a_hbm, c_hbm,
                   ls_v, ids_v, st_v, al_v, idx_v, rows_v, acc_a, acc_c, sem):
    wid = lax.axis_index("s") * _NC + lax.axis_index("c")
    pltpu.sync_copy(ls_hbm, ls_v)
    pltpu.sync_copy(ids_hbm, ids_v)
    pltpu.sync_copy(st_hbm, st_v)
    pltpu.sync_copy(al_hbm, al_v)
    lanes = lax.iota(jnp.int32, _L)
    seeds = jnp.minimum(wid * _SEEDS_PER_W + lanes, _S - 1)
    m2 = lanes < _SEEDS_PER_W
    idsg = plsc.load_gather(ids_v, [seeds], mask=m2)
    rowidx = idsg * (_S // 2) + wid
    plsc.store_scatter(idx_v, [lanes], rowidx, mask=m2)
    pltpu.async_copy(bw2_hbm.at[idx_v], rows_v, sem).wait()
    one = jnp.ones((_L,), jnp.float32)
    zero = jnp.zeros((_L,), jnp.float32)
    for sl in range(_SEEDS_PER_W):
        sval = jnp.full((_L,), wid * _SEEDS_PER_W + sl, jnp.int32)
        al_s = plsc.load_gather(al_v, [sval])
        st_s = plsc.load_gather(st_v, [sval])
        ls_s = plsc.load_gather(ls_v, [sval])
        active = (ls_s >= 3) & (ls_s <= 6)
        is0 = active & (st_s == 0)
        is1 = active & (st_s == 1)
        is2 = active & (st_s != 0) & (st_s != 1)
        for g in range(_CHUNK // _L):
            w = rows_v[sl, pl.ds(sl * _CHUNK + g * _L, _L)]
            a = jnp.where(is0, al_s * w + (one - al_s), jnp.where(is1, w, one))
            c = jnp.where(is2, w, zero)
            acc_a[pl.ds(sl * _CHUNK + g * _L, _L)] = a
            acc_c[pl.ds(sl * _CHUNK + g * _L, _L)] = c
    span = _SEEDS_PER_W * _CHUNK
    pltpu.sync_copy(acc_a, a_hbm.at[pl.ds(wid * span, span)])
    pltpu.sync_copy(acc_c, c_hbm.at[pl.ds(wid * span, span)])


def _sc_coeffs(ls, ids, st, al, bw2):
    H = _S * _CHUNK
    mesh = plsc.VectorSubcoreMesh(core_axis_name="c", subcore_axis_name="s",
                                  num_cores=_NC, num_subcores=_NS)
    span = _SEEDS_PER_W * _CHUNK
    return pl.kernel(
        _sc_coeff_body,
        out_type=(jax.ShapeDtypeStruct((H,), jnp.float32),
                  jax.ShapeDtypeStruct((H,), jnp.float32)),
        mesh=mesh,
        compiler_params=pltpu.CompilerParams(needs_layout_passes=False),
        scratch_types=[
            pltpu.VMEM((_S,), jnp.int32),
            pltpu.VMEM((_S,), jnp.int32),
            pltpu.VMEM((_S,), jnp.int32),
            pltpu.VMEM((_S,), jnp.float32),
            pltpu.VMEM((_SEEDS_PER_W,), jnp.int32),
            pltpu.VMEM((_SEEDS_PER_W, _SEEDS_PER_W * _CHUNK), jnp.float32),
            pltpu.VMEM((span,), jnp.float32),
            pltpu.VMEM((span,), jnp.float32),
            pltpu.SemaphoreType.DMA,
        ],
    )(ls, ids, st, al, bw2)


def _tc_blend_body(a_ref, c_ref, x_ref, o_ref):
    o_ref[...] = x_ref[...] * a_ref[...] + c_ref[...]


def kernel(x, lifecycle_states, blueprint_ids, grafting_strategies,
           blend_factors, blueprint_weights):
    B, H = x.shape
    bw2 = blueprint_weights.reshape(_NB * (_S // 2), _SEEDS_PER_W * _CHUNK)
    a, c = _sc_coeffs(lifecycle_states, blueprint_ids, grafting_strategies,
                      blend_factors, bw2)
    a2 = a.reshape(1, H)
    c2 = c.reshape(1, H)
    R = 512
    grid = (B // R,)
    row = lambda: pl.BlockSpec((1, H), lambda i: (0, 0))
    return pl.pallas_call(
        _tc_blend_body,
        grid=grid,
        in_specs=[row(), row(), pl.BlockSpec((R, H), lambda i: (i, 0))],
        out_specs=pl.BlockSpec((R, H), lambda i: (i, 0)),
        out_shape=jax.ShapeDtypeStruct((B, H), x.dtype),
    )(a2, c2, x)
